# baseline (speedup 1.0000x reference)
import functools

import jax
import jax.numpy as jnp
from jax import lax
from jax.experimental import pallas as pl
from jax.experimental.pallas import tpu as pltpu
from jax.experimental.pallas import tpu_sc as plsc

BSZ = 64
BEAM = 8
VOCAB = 100000
EOS = 2
CW = 2560
NFW = VOCAB // CW
TAILC = VOCAB - NFW * CW
BLKC = 512
BPW = CW // BLKC
FBPB = VOCAB // BLKC
BPB = FBPB + 1
NBLK = BEAM * BPB
TAILV = TAILC // 16
VPB = BLKC // 16
K = 16
NC, NS = 2, 16
NW = NC * NS
RPW = BSZ // NW
L1N = 1600
L2N = 112


def _sc_body(lp_hbm, bias_hbm, src_hbm, step_hbm,
             outs_hbm, outi_hbm, outb_hbm,
             win0_v, win1_v, tail_v, l1_v, l2_v, cand_v, l1c_v, l2c_v,
             bias_v, src_v, step_v, sel_v,
             outs_v, outi_v, outb_v,
             sem0, sem1, semg):
  cid = lax.axis_index("c")
  sid = lax.axis_index("s")
  wid = sid * NC + cid
  minf = jnp.float32(-jnp.inf)
  iota = lax.iota(jnp.int32, 16)
  lane0 = iota == 0

  pltpu.sync_copy(src_hbm, src_v)
  pltpu.sync_copy(step_hbm, step_v)
  step = step_v[pl.ds(0, 16)][0]

  def sread(ref, i):
    return plsc.load_gather(ref, [jnp.full((16,), i, jnp.int32)])[0]

  def sstore(ref, i, val):
    plsc.store_scatter(ref, [jnp.full((16,), i, jnp.int32)],
                       jnp.full((16,), val, ref.dtype), mask=lane0)

  def ffs_scal(mask_vec):
    return plsc.all_reduce_ffs(mask_vec)[0]

  def vmaxn(load, n):
    na = min(4, n)
    accs = [load(v) for v in range(na)]
    for v in range(na, n):
      accs[v % na] = jnp.maximum(accs[v % na], load(v))
    while len(accs) > 1:
      accs = [jnp.maximum(accs[i], accs[i + 1])
              for i in range(0, len(accs) - 1, 2)] + (
                  [accs[-1]] if len(accs) % 2 else [])
    return accs[0]

  def argmax_ref(ref, nv):
    vs = [ref[pl.ds(j * 16, 16)] for j in range(nv)]
    m = vs[0]
    for j in range(1, nv):
      m = jnp.maximum(m, vs[j])
    gm = jnp.max(m)
    pos = jnp.int32(16 * nv)
    for j in reversed(range(nv)):
      fj = ffs_scal(vs[j] == gm)
      pos = jnp.where(fj < 16, j * 16 + fj, pos)
    return gm, pos

  @pl.loop(0, RPW)
  def _row(ri):
    r = wid * RPW + ri
    rb0 = r * BEAM
    pltpu.sync_copy(bias_hbm.at[r], bias_v)
    bias_vec = bias_v[pl.ds(0, 16)]
    src_len = sread(src_v, r)
    min_len = 0 * src_len + 1
    max_len = 2 * src_len + 10
    eos_ninf = jnp.logical_or(step < min_len, step > max_len)
    eos_zero = step == max_len

    def start_win(w, buf, sem):
      off = pl.multiple_of(w * CW, 128)
      pltpu.make_async_copy(
          lp_hbm.at[pl.ds(rb0, BEAM), pl.ds(off, CW)], buf, sem).start()

    def wait_win(w, buf, sem):
      off = pl.multiple_of(w * CW, 128)
      pltpu.make_async_copy(
          lp_hbm.at[pl.ds(rb0, BEAM), pl.ds(off, CW)], buf, sem).wait()

    def compute_win(w, buf):
      for beam_ in range(BEAM):
        @pl.loop(0, BPW, unroll=BPW)
        def _blk(b):
          acc = vmaxn(lambda v: buf[beam_, pl.ds(b * BLKC + v * 16, 16)],
                      VPB)
          sstore(l1_v, beam_ * BPB + w * BPW + b,
                 jnp.max(acc) + bias_vec[beam_])

    start_win(0, win0_v, sem0)
    start_win(1, win1_v, sem1)

    @pl.loop(0, (NFW - 1) // 2)
    def _wpair(p):
      w0 = 2 * p
      w1 = w0 + 1
      wait_win(w0, win0_v, sem0)
      compute_win(w0, win0_v)
      start_win(w0 + 2, win0_v, sem0)
      wait_win(w1, win1_v, sem1)
      compute_win(w1, win1_v)
      @pl.when(w1 + 2 < NFW)
      def _():
        start_win(w1 + 2, win1_v, sem1)

    wait_win(NFW - 1, win0_v, sem0)
    compute_win(NFW - 1, win0_v)

    pltpu.sync_copy(
        lp_hbm.at[pl.ds(rb0, BEAM), pl.ds(NFW * CW, TAILC)], tail_v)
    for beam_ in range(BEAM):
      acc = vmaxn(lambda v: tail_v[beam_, pl.ds(v * 16, 16)], TAILV)
      sstore(l1_v, beam_ * BPB + FBPB, jnp.max(acc) + bias_vec[beam_])

    pltpu.sync_copy(
        lp_hbm.at[pl.ds(rb0, BEAM), pl.ds(0, BLKC)],
        win0_v.at[pl.ds(0, BEAM), pl.ds(0, BLKC)])
    for beam_ in range(BEAM):
      v0 = win0_v[beam_, pl.ds(0, 16)]
      v0m = jnp.where(eos_ninf, minf,
                      jnp.where(eos_zero, jnp.float32(0.0), v0))
      v0 = jnp.where(iota == EOS, v0m, v0)

      def load_eos(v, _v0=v0, _b=beam_):
        if v == 0:
          return _v0
        return win0_v[_b, pl.ds(v * 16, 16)]

      acc = vmaxn(load_eos, VPB)
      sstore(l1_v, beam_ * BPB, jnp.max(acc) + bias_vec[beam_])

    l1_v[pl.ds(NBLK, 16)] = jnp.full((16,), minf, jnp.float32)
    l1_v[pl.ds(NBLK + 16, 16)] = jnp.full((16,), minf, jnp.float32)

    @pl.loop(0, L1N // 16)
    def _l2(i):
      sstore(l2_v, i, jnp.max(l1_v[pl.ds(i * 16, 16)]))
    t2 = l2_v[pl.ds(L2N - 16, 16)]
    l2_v[pl.ds(L2N - 16, 16)] = jnp.where(iota >= 4, minf, t2)

    @pl.loop(0, K)
    def _sel(t):
      gm, i2 = argmax_ref(l2_v, L2N // 16)
      vi = l1_v[pl.ds(i2 * 16, 16)]
      e = ffs_scal(vi == gm)
      sstore(sel_v, t, i2 * 16 + e)
      vi2 = jnp.where(iota == e, minf, vi)
      l1_v[pl.ds(i2 * 16, 16)] = vi2
      sstore(l2_v, i2, jnp.max(vi2))

    @pl.loop(0, K)
    def _gat(j):
      bid = sread(sel_v, j)
      beam = bid // BPB
      cb = bid % BPB
      dst_r = pl.multiple_of(j * 8, 8)
      src_r = pl.multiple_of(rb0 + 0 * beam, 8)

      @pl.when(cb < FBPB)
      def _full():
        off = pl.multiple_of(cb * BLKC, 128)
        pltpu.make_async_copy(
            lp_hbm.at[pl.ds(src_r, BEAM), pl.ds(off, BLKC)],
            cand_v.at[pl.ds(dst_r, BEAM), pl.ds(0, BLKC)], semg).start()
        pltpu.make_async_copy(
            lp_hbm.at[pl.ds(src_r, BEAM), pl.ds(off, BLKC)],
            cand_v.at[pl.ds(dst_r, BEAM), pl.ds(0, BLKC)], semg).wait()

      @pl.when(cb >= FBPB)
      def _tail():
        pltpu.make_async_copy(
            lp_hbm.at[pl.ds(src_r, BEAM), pl.ds(NFW * CW, TAILC)],
            tail_v, semg).start()
        pltpu.make_async_copy(
            lp_hbm.at[pl.ds(src_r, BEAM), pl.ds(NFW * CW, TAILC)],
            tail_v, semg).wait()
        row = j * 8 + beam
        for v in range(TAILV):
          cand_v[row, pl.ds(v * 16, 16)] = tail_v[beam, pl.ds(v * 16, 16)]

    @pl.loop(0, K)
    def _fix(j):
      bid = sread(sel_v, j)
      beam = bid // BPB
      cb = bid % BPB
      b = sread(bias_v, beam)
      is_b0 = cb == 0
      nv = jnp.where(cb < FBPB, VPB, TAILV)
      row = j * 8 + beam

      @pl.loop(0, VPB)
      def _v(v):
        x = cand_v[row, pl.ds(v * 16, 16)]
        xm = jnp.where(eos_ninf, minf,
                       jnp.where(eos_zero, jnp.float32(0.0), x))
        x = jnp.where((iota == EOS) & is_b0 & (v == 0), xm, x)
        x = x + b
        cand_v[row, pl.ds(v * 16, 16)] = x
        sstore(l1c_v, j * VPB + v, jnp.where(v < nv, jnp.max(x), minf))

    @pl.loop(0, K * VPB // 16)
    def _l2c(i):
      sstore(l2c_v, i, jnp.max(l1c_v[pl.ds(i * 16, 16)]))

    @pl.loop(0, K)
    def _out(t):
      gm, i2 = argmax_ref(l2c_v, K * VPB // 16 // 16)
      vi = l1c_v[pl.ds(i2 * 16, 16)]
      e = ffs_scal(vi == gm)
      q = i2 * 16 + e
      j = q // VPB
      v = q % VPB
      bid = sread(sel_v, j)
      beam = bid // BPB
      cb = bid % BPB
      row = j * 8 + beam
      x = cand_v[row, pl.ds(v * 16, 16)]
      lane = ffs_scal(x == gm)
      sstore(outs_v, t, gm)
      sstore(outb_v, t, beam)
      sstore(outi_v, t, cb * BLKC + v * 16 + lane)
      x2 = jnp.where(iota == lane, minf, x)
      cand_v[row, pl.ds(v * 16, 16)] = x2
      sstore(l1c_v, q, jnp.max(x2))
      vi2 = l1c_v[pl.ds(i2 * 16, 16)]
      sstore(l2c_v, i2, jnp.max(vi2))

    pltpu.sync_copy(outs_v, outs_hbm.at[r])
    pltpu.sync_copy(outi_v, outi_hbm.at[r])
    pltpu.sync_copy(outb_v, outb_hbm.at[r])


@functools.partial(
    pl.kernel,
    out_type=[
        jax.ShapeDtypeStruct((BSZ, K), jnp.float32),
        jax.ShapeDtypeStruct((BSZ, K), jnp.int32),
        jax.ShapeDtypeStruct((BSZ, K), jnp.int32),
    ],
    mesh=plsc.VectorSubcoreMesh(
        core_axis_name="c", subcore_axis_name="s",
        num_cores=NC, num_subcores=NS),
    compiler_params=pltpu.CompilerParams(needs_layout_passes=False),
    scratch_types=[
        pltpu.VMEM((BEAM, CW), jnp.float32),
        pltpu.VMEM((BEAM, CW), jnp.float32),
        pltpu.VMEM((BEAM, TAILC), jnp.float32),
        pltpu.VMEM((L1N,), jnp.float32),
        pltpu.VMEM((L2N,), jnp.float32),
        pltpu.VMEM((K * 8, BLKC), jnp.float32),
        pltpu.VMEM((K * VPB,), jnp.float32),
        pltpu.VMEM((K * VPB // 16,), jnp.float32),
        pltpu.VMEM((16,), jnp.float32),
        pltpu.VMEM((BSZ,), jnp.int32),
        pltpu.VMEM((16,), jnp.int32),
        pltpu.VMEM((16,), jnp.int32),
        pltpu.VMEM((16,), jnp.float32),
        pltpu.VMEM((16,), jnp.int32),
        pltpu.VMEM((16,), jnp.int32),
        pltpu.SemaphoreType.DMA,
        pltpu.SemaphoreType.DMA,
        pltpu.SemaphoreType.DMA,
    ],
)
def _sc_kernel(*args):
  _sc_body(*args)


def kernel(lprobs, scores, src_lengths, step):
  lp = lprobs.reshape(BSZ * BEAM, VOCAB)
  step_i = jnp.asarray(step, jnp.int32)
  bias = lax.dynamic_index_in_dim(scores, step_i - 1, axis=2, keepdims=False)
  bias16 = jnp.concatenate(
      [bias.astype(jnp.float32), jnp.zeros((BSZ, 8), jnp.float32)], axis=1)
  src32 = src_lengths.astype(jnp.int32)
  step_arr = jnp.full((16,), step_i, jnp.int32)
  scores_buf, indices_buf, beams_buf = _sc_kernel(lp, bias16, src32, step_arr)
  return scores_buf, indices_buf, beams_buf


# speedup vs baseline: 1.8188x; 1.8188x over previous
import functools

import jax
import jax.numpy as jnp
from jax import lax
from jax.experimental import pallas as pl
from jax.experimental.pallas import tpu as pltpu
from jax.experimental.pallas import tpu_sc as plsc

BSZ = 64
BEAM = 8
VOCAB = 100000
EOS = 2
CW = 2560
NFW = VOCAB // CW
TAILC = VOCAB - NFW * CW
BLKC = 512
BPW = CW // BLKC
FBPB = VOCAB // BLKC
BPB = FBPB + 1
NBLK = BEAM * BPB
TAILV = TAILC // 16
VPB = BLKC // 16
K = 16
NC, NS = 2, 16
NW = NC * NS
RPW = BSZ // NW
L1B = 1600
L2G = L1B // 16
L2B = 112


def _sc_body(lp_hbm, bias_hbm, src_hbm, step_hbm,
             outs_hbm, outi_hbm, outb_hbm,
             win0_v, win1_v, tail_v, l1_v, l2_v, l3_v,
             cand_v, ctmp_v, l1c_v, l2c_v,
             bias_v, src_v, step_v, sel_v,
             outs_v, outi_v, outb_v,
             sem0, sem1, semg):
  cid = lax.axis_index("c")
  sid = lax.axis_index("s")
  wid = sid * NC + cid
  minf = jnp.float32(-jnp.inf)
  minf_vec = jnp.full((16,), minf, jnp.float32)
  iota = lax.iota(jnp.int32, 16)
  lane0 = iota == 0

  pltpu.sync_copy(src_hbm, src_v)
  pltpu.sync_copy(step_hbm, step_v)
  step = step_v[pl.ds(0, 16)][0]

  def sread(ref, i):
    return plsc.load_gather(ref, [jnp.full((16,), i, jnp.int32)])[0]

  def sstore(ref, i, val):
    plsc.store_scatter(ref, [jnp.full((16,), i, jnp.int32)],
                       jnp.full((16,), val, ref.dtype), mask=lane0)

  def ffs_scal(mask_vec):
    return plsc.all_reduce_ffs(mask_vec)[0]

  def vmaxn(load, n):
    na = min(4, n)
    accs = [load(v) for v in range(na)]
    for v in range(na, n):
      accs[v % na] = jnp.maximum(accs[v % na], load(v))
    while len(accs) > 1:
      accs = [jnp.maximum(accs[i], accs[i + 1])
              for i in range(0, len(accs) - 1, 2)] + (
                  [accs[-1]] if len(accs) % 2 else [])
    return accs[0]

  def argmax_ref(ref, nv):
    vs = [ref[pl.ds(j * 16, 16)] for j in range(nv)]
    m = vs[0]
    for j in range(1, nv):
      m = jnp.maximum(m, vs[j])
    gm = jnp.max(m)
    pos = jnp.int32(16 * nv)
    for j in reversed(range(nv)):
      fj = ffs_scal(vs[j] == gm)
      pos = jnp.where(fj < 16, j * 16 + fj, pos)
    return gm, pos

  @pl.loop(0, RPW)
  def _row(ri):
    r = wid * RPW + ri
    rb0 = r * BEAM
    pltpu.sync_copy(bias_hbm.at[r], bias_v)
    bias_vec = bias_v[pl.ds(0, 16)]
    src_len = sread(src_v, r)
    min_len = 0 * src_len + 1
    max_len = 2 * src_len + 10
    eos_ninf = jnp.logical_or(step < min_len, step > max_len)
    eos_zero = step == max_len

    def start_win(w, buf, sem):
      off = pl.multiple_of(w * CW, 128)
      pltpu.make_async_copy(
          lp_hbm.at[pl.ds(rb0, BEAM), pl.ds(off, CW)], buf, sem).start()

    def wait_win(w, buf, sem):
      off = pl.multiple_of(w * CW, 128)
      pltpu.make_async_copy(
          lp_hbm.at[pl.ds(rb0, BEAM), pl.ds(off, CW)], buf, sem).wait()

    def compute_win(w, buf):
      for beam_ in range(BEAM):
        bsp = jnp.full((16,), bias_vec[beam_], jnp.float32)

        @pl.loop(0, BPW)
        def _blk(b):
          acc = vmaxn(lambda v: buf[beam_, pl.ds(b * BLKC + v * 16, 16)],
                      VPB)
          g = beam_ * BPB + w * BPW + b
          l1_v[pl.ds(g * 16, 16)] = acc + bsp

    start_win(0, win0_v, sem0)
    start_win(1, win1_v, sem1)

    @pl.loop(0, (NFW - 1) // 2)
    def _wpair(p):
      w0 = 2 * p
      w1 = w0 + 1
      wait_win(w0, win0_v, sem0)
      compute_win(w0, win0_v)
      start_win(w0 + 2, win0_v, sem0)
      wait_win(w1, win1_v, sem1)
      compute_win(w1, win1_v)
      @pl.when(w1 + 2 < NFW)
      def _():
        start_win(w1 + 2, win1_v, sem1)

    wait_win(NFW - 1, win0_v, sem0)
    compute_win(NFW - 1, win0_v)

    pltpu.sync_copy(
        lp_hbm.at[pl.ds(rb0, BEAM), pl.ds(NFW * CW, TAILC)], tail_v)
    for beam_ in range(BEAM):
      acc = vmaxn(lambda v: tail_v[beam_, pl.ds(v * 16, 16)], TAILV)
      g = beam_ * BPB + FBPB
      l1_v[pl.ds(g * 16, 16)] = acc + jnp.full(
          (16,), bias_vec[beam_], jnp.float32)

    pltpu.sync_copy(
        lp_hbm.at[pl.ds(rb0, BEAM), pl.ds(0, BLKC)],
        win0_v.at[pl.ds(0, BEAM), pl.ds(0, BLKC)])
    for beam_ in range(BEAM):
      v0 = win0_v[beam_, pl.ds(0, 16)]
      v0m = jnp.where(eos_ninf, minf,
                      jnp.where(eos_zero, jnp.float32(0.0), v0))
      v0 = jnp.where(iota == EOS, v0m, v0)

      def load_eos(v, _v0=v0, _b=beam_):
        if v == 0:
          return _v0
        return win0_v[_b, pl.ds(v * 16, 16)]

      acc = vmaxn(load_eos, VPB)
      l1_v[pl.ds(beam_ * BPB * 16, 16)] = acc + jnp.full(
          (16,), bias_vec[beam_], jnp.float32)

    @pl.loop(NBLK, L1B)
    def _p1(i):
      l1_v[pl.ds(i * 16, 16)] = minf_vec

    @pl.loop(0, L2G)
    def _l2t(g):
      m = vmaxn(lambda i: l1_v[pl.ds(g * 256 + i * 16, 16)], 16)
      l2_v[pl.ds(g * 16, 16)] = m

    @pl.loop(L2G, L2B)
    def _p2(g):
      l2_v[pl.ds(g * 16, 16)] = minf_vec

    for g in range(L2B // 16):
      m = vmaxn(lambda i, _g=g: l2_v[pl.ds(_g * 256 + i * 16, 16)], 16)
      l3_v[pl.ds(g * 16, 16)] = m

    @pl.loop(0, K)
    def _sel(t):
      gm, pos3 = argmax_ref(l3_v, L2B // 16)
      g3 = pos3 // 16
      ln = pos3 % 16
      idx2 = g3 * 256 + iota * 16 + ln
      v2 = plsc.load_gather(l2_v, [idx2])
      j2 = g3 * 16 + ffs_scal(v2 == gm)
      idx1 = j2 * 256 + iota * 16 + ln
      v1 = plsc.load_gather(l1_v, [idx1])
      b = j2 * 16 + ffs_scal(v1 == gm)
      sstore(sel_v, t, b)
      vb = l1_v[pl.ds(b * 16, 16)]
      l1_v[pl.ds(b * 16, 16)] = jnp.where(iota == ln, minf, vb)
      m1 = jnp.max(plsc.load_gather(l1_v, [idx1]))
      v2v = l2_v[pl.ds(j2 * 16, 16)]
      l2_v[pl.ds(j2 * 16, 16)] = jnp.where(iota == ln, m1, v2v)
      m2 = jnp.max(plsc.load_gather(l2_v, [idx2]))
      v3v = l3_v[pl.ds(g3 * 16, 16)]
      l3_v[pl.ds(g3 * 16, 16)] = jnp.where(iota == ln, m2, v3v)

    selvec = sel_v[pl.ds(0, 16)]

    @pl.loop(0, K)
    def _gat(j):
      bid = sread(sel_v, j)
      dup = ffs_scal((selvec == bid) & (iota < j)) < 16
      beam = bid // BPB
      cb = bid % BPB

      @pl.when(jnp.logical_not(dup))
      def _ndup():
        @pl.when(cb < FBPB)
        def _full():
          off = pl.multiple_of(cb * BLKC, 128)
          pltpu.make_async_copy(
              lp_hbm.at[pl.ds(rb0, BEAM), pl.ds(off, BLKC)],
              ctmp_v, semg).start()
          pltpu.make_async_copy(
              lp_hbm.at[pl.ds(rb0, BEAM), pl.ds(off, BLKC)],
              ctmp_v, semg).wait()
          for v in range(VPB):
            cand_v[j, pl.ds(v * 16, 16)] = ctmp_v[beam, pl.ds(v * 16, 16)]

        @pl.when(cb >= FBPB)
        def _tail():
          for v in range(TAILV):
            cand_v[j, pl.ds(v * 16, 16)] = tail_v[beam, pl.ds(v * 16, 16)]

    @pl.loop(0, K)
    def _fix(j):
      bid = sread(sel_v, j)
      dup = ffs_scal((selvec == bid) & (iota < j)) < 16
      beam = bid // BPB
      cb = bid % BPB

      @pl.when(jnp.logical_not(dup))
      def _ndup():
        bsc = sread(bias_v, beam)
        is_b0 = cb == 0
        nv = jnp.where(cb < FBPB, VPB, TAILV)

        @pl.loop(0, VPB)
        def _v(v):
          x = cand_v[j, pl.ds(v * 16, 16)]
          xm = jnp.where(eos_ninf, minf,
                         jnp.where(eos_zero, jnp.float32(0.0), x))
          x = jnp.where((iota == EOS) & is_b0 & (v == 0), xm, x)
          x = x + bsc
          cand_v[j, pl.ds(v * 16, 16)] = x
          sstore(l1c_v, j * VPB + v, jnp.where(v < nv, jnp.max(x), minf))

      @pl.when(dup)
      def _dup():
        l1c_v[pl.ds(j * VPB, 16)] = minf_vec
        l1c_v[pl.ds(j * VPB + 16, 16)] = minf_vec

    @pl.loop(0, K * VPB // 16)
    def _l2c(i):
      sstore(l2c_v, i, jnp.max(l1c_v[pl.ds(i * 16, 16)]))

    @pl.loop(0, K)
    def _out(t):
      gm, i2 = argmax_ref(l2c_v, K * VPB // 16 // 16)
      vi = l1c_v[pl.ds(i2 * 16, 16)]
      e = ffs_scal(vi == gm)
      q = i2 * 16 + e
      j = q // VPB
      v = q % VPB
      bid = sread(sel_v, j)
      beam = bid // BPB
      cb = bid % BPB
      x = cand_v[j, pl.ds(v * 16, 16)]
      lane = ffs_scal(x == gm)
      sstore(outs_v, t, gm)
      sstore(outb_v, t, beam)
      sstore(outi_v, t, cb * BLKC + v * 16 + lane)
      x2 = jnp.where(iota == lane, minf, x)
      cand_v[j, pl.ds(v * 16, 16)] = x2
      sstore(l1c_v, q, jnp.max(x2))
      vi2 = l1c_v[pl.ds(i2 * 16, 16)]
      sstore(l2c_v, i2, jnp.max(vi2))

    pltpu.sync_copy(outs_v, outs_hbm.at[r])
    pltpu.sync_copy(outi_v, outi_hbm.at[r])
    pltpu.sync_copy(outb_v, outb_hbm.at[r])


@functools.partial(
    pl.kernel,
    out_type=[
        jax.ShapeDtypeStruct((BSZ, K), jnp.float32),
        jax.ShapeDtypeStruct((BSZ, K), jnp.int32),
        jax.ShapeDtypeStruct((BSZ, K), jnp.int32),
    ],
    mesh=plsc.VectorSubcoreMesh(
        core_axis_name="c", subcore_axis_name="s",
        num_cores=NC, num_subcores=NS),
    compiler_params=pltpu.CompilerParams(needs_layout_passes=False),
    scratch_types=[
        pltpu.VMEM((BEAM, CW), jnp.float32),
        pltpu.VMEM((BEAM, CW), jnp.float32),
        pltpu.VMEM((BEAM, TAILC), jnp.float32),
        pltpu.VMEM((L1B * 16,), jnp.float32),
        pltpu.VMEM((L2B * 16,), jnp.float32),
        pltpu.VMEM((L2B,), jnp.float32),
        pltpu.VMEM((K, BLKC), jnp.float32),
        pltpu.VMEM((BEAM, BLKC), jnp.float32),
        pltpu.VMEM((K * VPB,), jnp.float32),
        pltpu.VMEM((K * VPB // 16,), jnp.float32),
        pltpu.VMEM((16,), jnp.float32),
        pltpu.VMEM((BSZ,), jnp.int32),
        pltpu.VMEM((16,), jnp.int32),
        pltpu.VMEM((16,), jnp.int32),
        pltpu.VMEM((16,), jnp.float32),
        pltpu.VMEM((16,), jnp.int32),
        pltpu.VMEM((16,), jnp.int32),
        pltpu.SemaphoreType.DMA,
        pltpu.SemaphoreType.DMA,
        pltpu.SemaphoreType.DMA,
    ],
)
def _sc_kernel(*args):
  _sc_body(*args)


def kernel(lprobs, scores, src_lengths, step):
  lp = lprobs.reshape(BSZ * BEAM, VOCAB)
  step_i = jnp.asarray(step, jnp.int32)
  bias = lax.dynamic_index_in_dim(scores, step_i - 1, axis=2, keepdims=False)
  bias16 = jnp.concatenate(
      [bias.astype(jnp.float32), jnp.zeros((BSZ, 8), jnp.float32)], axis=1)
  src32 = src_lengths.astype(jnp.int32)
  step_arr = jnp.full((16,), step_i, jnp.int32)
  scores_buf, indices_buf, beams_buf = _sc_kernel(lp, bias16, src32, step_arr)
  return scores_buf, indices_buf, beams_buf
